# baseline (speedup 1.0000x reference)
import functools

import jax
import jax.numpy as jnp
from jax import lax
from jax.experimental import pallas as pl
from jax.experimental.pallas import tpu as pltpu
from jax.experimental.pallas import tpu_sc as plsc

D = 128
L = 200
LANES = 16
GROUPS = D // LANES
NBUF = 4
C = 40


@functools.lru_cache(maxsize=None)
def _build(B, V):
    NW = 32
    P = (B * L) // NW
    NCH = P // C

    mesh = plsc.VectorSubcoreMesh(core_axis_name="c", subcore_axis_name="s")

    @functools.partial(
        pl.kernel,
        mesh=mesh,
        out_type=jax.ShapeDtypeStruct((B * L, D), jnp.float32),
        scratch_types=[
            pltpu.VMEM((P,), jnp.int32),
            pltpu.VMEM((L, D), jnp.float32),
        ]
        + [pltpu.VMEM((C, D), jnp.float32) for _ in range(NBUF)]
        + [pltpu.SemaphoreType.DMA for _ in range(2 * NBUF)],
    )
    def emb_pe(x_hbm, table_hbm, pe_hbm, out_hbm, idx_v, pe_v, *rest):
        bufs = rest[:NBUF]
        gsems = rest[NBUF:2 * NBUF]
        ssems = rest[2 * NBUF:]

        wid = lax.axis_index("s") * 2 + lax.axis_index("c")
        p0 = wid * P
        pltpu.sync_copy(pe_hbm, pe_v)
        pltpu.sync_copy(x_hbm.at[pl.ds(p0, P)], idx_v)

        def gather(j, b):
            return pltpu.make_async_copy(
                table_hbm.at[idx_v.at[pl.ds(j * C, C)]], bufs[b], gsems[b])

        def store(j, b):
            return pltpu.make_async_copy(
                bufs[b], out_hbm.at[pl.ds(p0 + j * C, C)], ssems[b])

        for k in range(NBUF - 1):
            gather(k, k).start()

        def body(j2, carry):
            for b in range(NBUF):
                pb = (b - 1) % NBUF
                j = j2 * NBUF + b

                @pl.when(j > 0)
                def _():
                    store(j - 1, pb).wait()

                @pl.when(j + NBUF - 1 < NCH)
                def _():
                    gather(j + NBUF - 1, pb).start()

                gather(j, b).wait()
                po = lax.rem(j * C, L)

                def add_row(r, c):
                    for g in range(GROUPS):
                        sl = pl.ds(g * LANES, LANES)
                        bufs[b][r, sl] = bufs[b][r, sl] + pe_v[po + r, sl]
                    return c

                lax.fori_loop(0, C, add_row, 0)
                store(j, b).start()
            return carry

        lax.fori_loop(0, NCH // NBUF, body, 0)
        store(NCH - 1, (NCH - 1) % NBUF).wait()

    return emb_pe


def kernel(x, table, pe):
    B, Lx = x.shape
    xi = x.reshape(-1).astype(jnp.int32)
    pef = pe.reshape(Lx, D)
    out = _build(B, table.shape[0])(xi, table, pef)
    return out.reshape(B, Lx, D)


# speedup vs baseline: 1.0426x; 1.0426x over previous
import functools

import jax
import jax.numpy as jnp
from jax import lax
from jax.experimental import pallas as pl
from jax.experimental.pallas import tpu as pltpu
from jax.experimental.pallas import tpu_sc as plsc

D = 128
L = 200
LANES = 16
GROUPS = D // LANES
NBUF = 2
C = 128
NS = 16


@functools.lru_cache(maxsize=None)
def _build(B, V):
    NW = 32
    P = (B * L) // NW
    NCH = P // C
    assert NCH % NBUF == 0 and P % C == 0 and C % 8 == 0

    mesh = plsc.VectorSubcoreMesh(core_axis_name="c", subcore_axis_name="s")

    @functools.partial(
        pl.kernel,
        mesh=mesh,
        out_type=jax.ShapeDtypeStruct((B * L, D), jnp.float32),
        scratch_types=[
            pltpu.VMEM((P,), jnp.int32),
            pltpu.VMEM((L, D), jnp.float32),
            pltpu.VMEM_SHARED((NS, NBUF, C, D), jnp.float32),
        ]
        + [pltpu.VMEM((C, D), jnp.float32) for _ in range(NBUF)]
        + [pltpu.SemaphoreType.DMA for _ in range(3 * NBUF)],
    )
    def emb_pe(x_hbm, table_hbm, pe_hbm, out_hbm, idx_v, pe_v, shared, *rest):
        bufs = rest[:NBUF]
        gsems = rest[NBUF:2 * NBUF]
        csems = rest[2 * NBUF:3 * NBUF]
        ssems = rest[3 * NBUF:]

        cid = lax.axis_index("c")
        sid = lax.axis_index("s")
        wid = sid * 2 + cid
        p0 = wid * P
        pltpu.sync_copy(pe_hbm, pe_v)
        pltpu.sync_copy(x_hbm.at[pl.ds(p0, P)], idx_v)

        def gather(j, b):
            return pltpu.make_async_copy(
                table_hbm.at[idx_v.at[pl.ds(j * C, C)]], bufs[b], gsems[b])

        def tocb(j, b):
            return pltpu.make_async_copy(
                bufs[b], shared.at[sid, b], csems[b])

        def store(j, b):
            return pltpu.make_async_copy(
                shared.at[sid, b], out_hbm.at[pl.ds(p0 + j * C, C)], ssems[b])

        for k in range(NBUF - 1):
            gather(k, k).start()

        def body(j2, carry):
            for b in range(NBUF):
                pb = (b - 1) % NBUF
                j = j2 * NBUF + b

                @pl.when(j > 0)
                def _():
                    tocb(j - 1, pb).wait()
                    store(j - 1, pb).start()

                @pl.when(j + NBUF - 1 < NCH)
                def _():
                    gather(j + NBUF - 1, pb).start()

                gather(j, b).wait()
                po = lax.rem(j * C, L)
                hi1 = lax.min(C, L - po)

                def add_row(r, c):
                    for g in range(GROUPS):
                        sl = pl.ds(g * LANES, LANES)
                        bufs[b][r, sl] = bufs[b][r, sl] + pe_v[po + r, sl]
                    return c

                def add_row_wrap(r, c):
                    for g in range(GROUPS):
                        sl = pl.ds(g * LANES, LANES)
                        bufs[b][r, sl] = bufs[b][r, sl] + pe_v[po + r - L, sl]
                    return c

                lax.fori_loop(0, hi1, add_row, 0)
                lax.fori_loop(hi1, C, add_row_wrap, 0)

                @pl.when(j >= NBUF)
                def _():
                    store(j - NBUF, b).wait()

                tocb(j, b).start()
            return carry

        lax.fori_loop(0, NCH // NBUF, body, 0)
        lb = (NCH - 1) % NBUF
        tocb(NCH - 1, lb).wait()
        store(NCH - 1, lb).start()
        store(NCH - 1, lb).wait()
        store(NCH - 2, (NCH - 2) % NBUF).wait()

    return emb_pe


def kernel(x, table, pe):
    B, Lx = x.shape
    xi = x.reshape(-1).astype(jnp.int32)
    pef = pe.reshape(Lx, D)
    out = _build(B, table.shape[0])(xi, table, pef)
    return out.reshape(B, Lx, D)


# speedup vs baseline: 3.6896x; 3.5388x over previous
import functools

import jax
import jax.numpy as jnp
from jax import lax
from jax.experimental import pallas as pl
from jax.experimental.pallas import tpu as pltpu
from jax.experimental.pallas import tpu_sc as plsc

D = 128
L = 200
LANES = 16
GROUPS = D // LANES
NBUF = 2
C = 200
H0 = 104
H1 = C - H0


@functools.lru_cache(maxsize=None)
def _build(B, V):
    NW = 32
    P = (B * L) // NW
    NCH = P // C
    assert NCH % NBUF == 0 and P % C == 0 and L == C

    mesh = plsc.VectorSubcoreMesh(core_axis_name="c", subcore_axis_name="s")

    @functools.partial(
        pl.kernel,
        mesh=mesh,
        out_type=jax.ShapeDtypeStruct((B * L, D), jnp.float32),
        scratch_types=[
            pltpu.VMEM((P,), jnp.int32),
            pltpu.VMEM((L, D), jnp.float32),
        ]
        + [pltpu.VMEM((C, D), jnp.float32) for _ in range(NBUF)]
        + [pltpu.SemaphoreType.DMA for _ in range(2 * NBUF)],
    )
    def emb_pe(x_hbm, table_hbm, pe_hbm, out_hbm, idx_v, pe_v, *rest):
        bufs = rest[:NBUF]
        gsems = rest[NBUF:2 * NBUF]
        ssems = rest[2 * NBUF:]

        wid = lax.axis_index("s") * 2 + lax.axis_index("c")
        p0 = wid * P
        pltpu.sync_copy(pe_hbm, pe_v)
        pltpu.sync_copy(x_hbm.at[pl.ds(p0, P)], idx_v)

        def gather(j, b):
            return pltpu.make_async_copy(
                table_hbm.at[idx_v.at[pl.ds(j * C, C)]], bufs[b], gsems[b])

        def store_half(j, b, h):
            off, n = (0, H0) if h == 0 else (H0, H1)
            return pltpu.make_async_copy(
                bufs[b].at[pl.ds(off, n)],
                out_hbm.at[pl.ds(p0 + j * C + off, n)], ssems[b])

        def store_wait(b):
            return pltpu.make_async_copy(
                bufs[b], out_hbm.at[pl.ds(p0, C)], ssems[b])

        for k in range(NBUF - 1):
            gather(k, k).start()

        def add_rows(b, lo, hi):
            def add_row(r, c):
                for g in range(GROUPS):
                    sl = pl.ds(g * LANES, LANES)
                    plsc.addupdate(bufs[b].at[r, sl], pe_v[r, sl])
                return c

            lax.fori_loop(lo, hi, add_row, 0)

        def body(j2, carry):
            for b in range(NBUF):
                pb = (b - 1) % NBUF
                j = j2 * NBUF + b

                @pl.when(j > 0)
                def _():
                    store_wait(pb).wait()

                @pl.when(j + NBUF - 1 < NCH)
                def _():
                    gather(j + NBUF - 1, pb).start()

                gather(j, b).wait()
                add_rows(b, 0, H0)
                store_half(j, b, 0).start()
                add_rows(b, H0, C)
                store_half(j, b, 1).start()
            return carry

        lax.fori_loop(0, NCH // NBUF, body, 0)
        store_wait((NCH - 1) % NBUF).wait()

    return emb_pe


def kernel(x, table, pe):
    B, Lx = x.shape
    xi = x.reshape(-1).astype(jnp.int32)
    pef = pe.reshape(Lx, D)
    out = _build(B, table.shape[0])(xi, table, pef)
    return out.reshape(B, Lx, D)


# speedup vs baseline: 3.7778x; 1.0239x over previous
import functools

import jax
import jax.numpy as jnp
from jax import lax
from jax.experimental import pallas as pl
from jax.experimental.pallas import tpu as pltpu
from jax.experimental.pallas import tpu_sc as plsc

D = 128
L = 200
LANES = 16
GROUPS = D // LANES
NBUF = 2
C = 200
SPLITS = (56, 48, 48, 48)


@functools.lru_cache(maxsize=None)
def _build(B, V):
    NW = 32
    P = (B * L) // NW
    NCH = P // C
    assert NCH % NBUF == 0 and P % C == 0 and L == C

    mesh = plsc.VectorSubcoreMesh(core_axis_name="c", subcore_axis_name="s")

    @functools.partial(
        pl.kernel,
        mesh=mesh,
        out_type=jax.ShapeDtypeStruct((B * L, D), jnp.float32),
        scratch_types=[
            pltpu.VMEM((P,), jnp.int32),
            pltpu.VMEM((L, D), jnp.float32),
        ]
        + [pltpu.VMEM((C, D), jnp.float32) for _ in range(NBUF)]
        + [pltpu.SemaphoreType.DMA for _ in range(2 * NBUF)],
    )
    def emb_pe(x_hbm, table_hbm, pe_hbm, out_hbm, idx_v, pe_v, *rest):
        bufs = rest[:NBUF]
        gsems = rest[NBUF:2 * NBUF]
        ssems = rest[2 * NBUF:]

        wid = lax.axis_index("s") * 2 + lax.axis_index("c")
        p0 = wid * P
        pltpu.sync_copy(pe_hbm, pe_v)
        pltpu.sync_copy(x_hbm.at[pl.ds(p0, P)], idx_v)

        def gather(j, b):
            return pltpu.make_async_copy(
                table_hbm.at[idx_v.at[pl.ds(j * C, C)]], bufs[b], gsems[b])

        offs = [sum(SPLITS[:k]) for k in range(len(SPLITS))]

        def store_piece(j, b, h):
            off, n = offs[h], SPLITS[h]
            return pltpu.make_async_copy(
                bufs[b].at[pl.ds(off, n)],
                out_hbm.at[pl.ds(p0 + j * C + off, n)], ssems[b])

        def store_wait(b):
            return pltpu.make_async_copy(
                bufs[b], out_hbm.at[pl.ds(p0, C)], ssems[b])

        for k in range(NBUF - 1):
            gather(k, k).start()

        def add_rows(b, lo, hi):
            def add_row(r, c):
                for g in range(GROUPS):
                    sl = pl.ds(g * LANES, LANES)
                    plsc.addupdate(bufs[b].at[r, sl], pe_v[r, sl])
                return c

            lax.fori_loop(lo, hi, add_row, 0)

        def body(j2, carry):
            for b in range(NBUF):
                pb = (b - 1) % NBUF
                j = j2 * NBUF + b

                @pl.when(j > 0)
                def _():
                    store_wait(pb).wait()

                @pl.when(j + NBUF - 1 < NCH)
                def _():
                    gather(j + NBUF - 1, pb).start()

                gather(j, b).wait()
                for h in range(len(SPLITS)):
                    add_rows(b, offs[h], offs[h] + SPLITS[h])
                    store_piece(j, b, h).start()
            return carry

        lax.fori_loop(0, NCH // NBUF, body, 0)
        store_wait((NCH - 1) % NBUF).wait()

    return emb_pe


def kernel(x, table, pe):
    B, Lx = x.shape
    xi = x.reshape(-1).astype(jnp.int32)
    pef = pe.reshape(Lx, D)
    out = _build(B, table.shape[0])(xi, table, pef)
    return out.reshape(B, Lx, D)
